# baseline (speedup 1.0000x reference)
import functools

import jax
import jax.numpy as jnp
from jax import lax
from jax.experimental import pallas as pl
from jax.experimental.pallas import tpu as pltpu
from jax.experimental.pallas import tpu_sc as plsc

_N_DAYS = 365
_N_BUCKETS = 1024
_N_RBF = 16
_N_BASES = 1 + _N_RBF
_N = 16384

_L = 16
_NC = 2
_NS = 16
_NW = _NC * _NS
_B_PER_W = _N // _NW
_CHUNK = 128
_N_CHUNKS = _B_PER_W // _CHUNK
_G_PER_CHUNK = _CHUNK // _L
_SR = 16
_SR_W = _SR * _N_BASES
_N_SROWS = _N_DAYS * _N_BUCKETS // _SR

_CENTERS = tuple(0.03125 + 0.0625 * i for i in range(_N_RBF))
_INV_SIGMA = 20.0

_mesh = plsc.VectorSubcoreMesh(core_axis_name="c", subcore_axis_name="s")


@functools.partial(
    pl.kernel,
    mesh=_mesh,
    compiler_params=pltpu.CompilerParams(
        needs_layout_passes=False, use_tc_tiling_on_sc=False
    ),
    out_type=jax.ShapeDtypeStruct((_N,), jnp.float32),
    scratch_types=[
        pltpu.VMEM((_B_PER_W,), jnp.float32),
        pltpu.VMEM((_B_PER_W,), jnp.int32),
        pltpu.VMEM((_B_PER_W,), jnp.int32),
        pltpu.VMEM((_B_PER_W,), jnp.int32),
        [pltpu.VMEM((_CHUNK,), jnp.int32) for _ in range(_N_CHUNKS)],
        [pltpu.VMEM((_CHUNK, _SR_W), jnp.float32) for _ in range(2)],
        pltpu.VMEM((_B_PER_W,), jnp.float32),
        [pltpu.SemaphoreType.DMA for _ in range(2)],
    ],
)
def _gauss_delta_sc(mvoc_hbm, day_hbm, bucket_hbm, w_hbm, out_hbm,
                    mvoc_v, day_v, bucket_v, idx_v, sidx_vs, srows_vs,
                    out_v, sems):
    wid = lax.axis_index("s") * _NC + lax.axis_index("c")
    base = wid * _B_PER_W

    pltpu.sync_copy(mvoc_hbm.at[pl.ds(base, _B_PER_W)], mvoc_v)
    pltpu.sync_copy(day_hbm.at[pl.ds(base, _B_PER_W)], day_v)
    pltpu.sync_copy(bucket_hbm.at[pl.ds(base, _B_PER_W)], bucket_v)

    for j in range(_N_CHUNKS):
        for g in range(_G_PER_CHUNK):
            o = j * _CHUNK + g * _L
            d = day_v[pl.ds(o, _L)]
            b = bucket_v[pl.ds(o, _L)]
            flat = d * _N_BUCKETS + b
            idx_v[pl.ds(o, _L)] = flat
            sidx_vs[j][pl.ds(g * _L, _L)] = flat >> 4

    def fire(j):
        return pltpu.async_copy(w_hbm.at[sidx_vs[j]], srows_vs[j % 2],
                                sems[j % 2])

    copies = [fire(0), fire(1)]

    for j in range(_N_CHUNKS):
        copies[j].wait()
        buf = srows_vs[j % 2]
        for g in range(_G_PER_CHUNK):
            o = j * _CHUNK + g * _L
            rows = lax.iota(jnp.int32, _L) + g * _L
            z = mvoc_v[pl.ds(o, _L)]
            z = jnp.minimum(jnp.maximum(z, 0.0), 1.0)
            off17 = (idx_v[pl.ds(o, _L)] & 15) * _N_BASES
            acc = plsc.load_gather(buf, [rows, off17])
            for k in range(_N_RBF):
                wk = plsc.load_gather(buf, [rows, off17 + (k + 1)])
                t = (z - _CENTERS[k]) * _INV_SIGMA
                acc = acc + jnp.exp(t * t * -0.5) * wk
            out_v[pl.ds(o, _L)] = acc
        if j + 2 < _N_CHUNKS:
            copies.append(fire(j + 2))

    pltpu.sync_copy(out_v, out_hbm.at[pl.ds(base, _B_PER_W)])


def kernel(mvoc, day_idx, bucket_idx, weights):
    w_sr = weights.reshape(_N_SROWS, _SR_W)
    out = _gauss_delta_sc(
        mvoc.reshape(_N),
        day_idx.reshape(_N),
        bucket_idx.reshape(_N),
        w_sr,
    )
    return out.reshape(_N, 1)


# speedup vs baseline: 6.3069x; 6.3069x over previous
import functools

import jax
import jax.numpy as jnp
from jax import lax
from jax.experimental import pallas as pl
from jax.experimental.pallas import tpu as pltpu
from jax.experimental.pallas import tpu_sc as plsc

_N_DAYS = 365
_N_BUCKETS = 1024
_N_RBF = 16
_N_BASES = 1 + _N_RBF
_N = 16384
_PLANE = _N_DAYS * _N_BUCKETS
_SEG_ROWS = _N_BASES * _PLANE // 16

_L = 16
_NC = 2
_NS = 16
_NW = _NC * _NS
_B_PER_W = _N // _NW
_CHUNK = 128
_N_CHUNKS = _B_PER_W // _CHUNK
_G_PER_CHUNK = _CHUNK // _L
_WIN = 7
_NSTREAM = 1 + _WIN

_INV_SIGMA = 20.0

_mesh = plsc.VectorSubcoreMesh(core_axis_name="c", subcore_axis_name="s")


@functools.partial(
    pl.kernel,
    mesh=_mesh,
    compiler_params=pltpu.CompilerParams(
        needs_layout_passes=False, use_tc_tiling_on_sc=False
    ),
    out_type=jax.ShapeDtypeStruct((_N,), jnp.float32),
    scratch_types=[
        pltpu.VMEM((_B_PER_W,), jnp.float32),
        pltpu.VMEM((_B_PER_W,), jnp.int32),
        pltpu.VMEM((_B_PER_W,), jnp.int32),
        pltpu.VMEM((_B_PER_W,), jnp.int32),
        pltpu.VMEM((_B_PER_W,), jnp.int32),
        pltpu.VMEM((_N_CHUNKS * _NSTREAM, _CHUNK), jnp.int32),
        [[pltpu.VMEM((_CHUNK, _L), jnp.float32)
          for _ in range(_NSTREAM)] for _ in range(2)],
        pltpu.VMEM((_B_PER_W,), jnp.float32),
        [pltpu.SemaphoreType.DMA for _ in range(2)],
    ],
)
def _gauss_delta_sc(mvoc_hbm, day_hbm, bucket_hbm, w_hbm, out_hbm,
                    mvoc_v, day_v, bucket_v, flat_v, win_v, sidx_v,
                    seg_vs, out_v, sems):
    wid = lax.axis_index("s") * _NC + lax.axis_index("c")
    base = wid * _B_PER_W

    pltpu.sync_copy(mvoc_hbm.at[pl.ds(base, _B_PER_W)], mvoc_v)
    pltpu.sync_copy(day_hbm.at[pl.ds(base, _B_PER_W)], day_v)
    pltpu.sync_copy(bucket_hbm.at[pl.ds(base, _B_PER_W)], bucket_v)

    for j in range(_N_CHUNKS):
        for g in range(_G_PER_CHUNK):
            o = j * _CHUNK + g * _L
            d = day_v[pl.ds(o, _L)]
            b = bucket_v[pl.ds(o, _L)]
            flat = d * _N_BUCKETS + b
            flat_v[pl.ds(o, _L)] = flat
            z = mvoc_v[pl.ds(o, _L)]
            z = jnp.minimum(jnp.maximum(z, 0.0), 1.0)
            m = (z * 16.0).astype(jnp.int32)
            s = jnp.minimum(jnp.maximum(m - 3, 0), 9)
            win_v[pl.ds(o, _L)] = s
            seg0 = flat >> 4
            for t in range(_NSTREAM):
                plane = (s + t) if t > 0 else jnp.zeros((_L,), jnp.int32)
                sidx_v[j * _NSTREAM + t, pl.ds(g * _L, _L)] = (
                    plane * (_PLANE // 16) + seg0)

    def fire(j):
        slot = j % 2
        return [pltpu.async_copy(w_hbm.at[sidx_v.at[j * _NSTREAM + t]],
                                 seg_vs[slot][t], sems[slot])
                for t in range(_NSTREAM)]

    copies = [fire(0), fire(1)]

    for j in range(_N_CHUNKS):
        for c in copies[j]:
            c.wait()
        bufs = seg_vs[j % 2]
        for g in range(_G_PER_CHUNK):
            o = j * _CHUNK + g * _L
            rows = lax.iota(jnp.int32, _L) + g * _L
            lane = flat_v[pl.ds(o, _L)] & 15
            z = mvoc_v[pl.ds(o, _L)]
            z = jnp.minimum(jnp.maximum(z, 0.0), 1.0)
            sf = win_v[pl.ds(o, _L)].astype(jnp.float32)
            acc = plsc.load_gather(bufs[0], [rows, lane])
            for t in range(1, _NSTREAM):
                wk = plsc.load_gather(bufs[t], [rows, lane])
                c = sf * 0.0625 + (0.03125 + (t - 1) * 0.0625)
                diff = (z - c) * _INV_SIGMA
                acc = acc + jnp.exp(diff * diff * -0.5) * wk
            out_v[pl.ds(o, _L)] = acc
        if j + 2 < _N_CHUNKS:
            copies.append(fire(j + 2))

    pltpu.sync_copy(out_v, out_hbm.at[pl.ds(base, _B_PER_W)])


def kernel(mvoc, day_idx, bucket_idx, weights):
    w_seg = weights.transpose(2, 0, 1).reshape(_SEG_ROWS, _L)
    out = _gauss_delta_sc(
        mvoc.reshape(_N),
        day_idx.reshape(_N),
        bucket_idx.reshape(_N),
        w_seg,
    )
    return out.reshape(_N, 1)


# speedup vs baseline: 6.4106x; 1.0165x over previous
import functools

import jax
import jax.numpy as jnp
from jax import lax
from jax.experimental import pallas as pl
from jax.experimental.pallas import tpu as pltpu
from jax.experimental.pallas import tpu_sc as plsc

_N_DAYS = 365
_N_BUCKETS = 1024
_N_RBF = 16
_N_BASES = 1 + _N_RBF
_N = 16384
_PLANE = _N_DAYS * _N_BUCKETS
_SEG_ROWS = _N_BASES * _PLANE // 16

_L = 16
_NC = 2
_NS = 16
_NW = _NC * _NS
_B_PER_W = _N // _NW
_CHUNK = 128
_N_CHUNKS = _B_PER_W // _CHUNK
_G_PER_CHUNK = _CHUNK // _L
_WIN = 6
_NSTREAM = 1 + _WIN

_INV_SIGMA = 20.0

_mesh = plsc.VectorSubcoreMesh(core_axis_name="c", subcore_axis_name="s")


@functools.partial(
    pl.kernel,
    mesh=_mesh,
    compiler_params=pltpu.CompilerParams(
        needs_layout_passes=False, use_tc_tiling_on_sc=False
    ),
    out_type=jax.ShapeDtypeStruct((_N,), jnp.float32),
    scratch_types=[
        pltpu.VMEM((_B_PER_W,), jnp.float32),
        pltpu.VMEM((_B_PER_W,), jnp.int32),
        pltpu.VMEM((_B_PER_W,), jnp.int32),
        pltpu.VMEM((_B_PER_W,), jnp.int32),
        pltpu.VMEM((_B_PER_W,), jnp.int32),
        pltpu.VMEM((_N_CHUNKS * _NSTREAM, _CHUNK), jnp.int32),
        [[pltpu.VMEM((_CHUNK, _L), jnp.float32)
          for _ in range(_NSTREAM)] for _ in range(2)],
        pltpu.VMEM((_B_PER_W,), jnp.float32),
        [pltpu.SemaphoreType.DMA for _ in range(2)],
    ],
)
def _gauss_delta_sc(mvoc_hbm, day_hbm, bucket_hbm, w_hbm, out_hbm,
                    mvoc_v, day_v, bucket_v, flat_v, win_v, sidx_v,
                    seg_vs, out_v, sems):
    wid = lax.axis_index("s") * _NC + lax.axis_index("c")
    base = wid * _B_PER_W

    pltpu.sync_copy(mvoc_hbm.at[pl.ds(base, _B_PER_W)], mvoc_v)
    pltpu.sync_copy(day_hbm.at[pl.ds(base, _B_PER_W)], day_v)
    pltpu.sync_copy(bucket_hbm.at[pl.ds(base, _B_PER_W)], bucket_v)

    for j in range(_N_CHUNKS):
        for g in range(_G_PER_CHUNK):
            o = j * _CHUNK + g * _L
            d = day_v[pl.ds(o, _L)]
            b = bucket_v[pl.ds(o, _L)]
            flat = d * _N_BUCKETS + b
            flat_v[pl.ds(o, _L)] = flat
            z = mvoc_v[pl.ds(o, _L)]
            z = jnp.minimum(jnp.maximum(z, 0.0), 1.0)
            m = (z * 16.0 - 0.5).astype(jnp.int32)
            s = jnp.minimum(jnp.maximum(m - 2, 0), 16 - _WIN)
            win_v[pl.ds(o, _L)] = s
            seg0 = flat >> 4
            for t in range(_NSTREAM):
                plane = (s + t) if t > 0 else jnp.zeros((_L,), jnp.int32)
                sidx_v[j * _NSTREAM + t, pl.ds(g * _L, _L)] = (
                    plane * (_PLANE // 16) + seg0)

    def fire(j):
        slot = j % 2
        return [pltpu.async_copy(w_hbm.at[sidx_v.at[j * _NSTREAM + t]],
                                 seg_vs[slot][t], sems[slot])
                for t in range(_NSTREAM)]

    copies = [fire(0), fire(1)]

    for j in range(_N_CHUNKS):
        for c in copies[j]:
            c.wait()
        bufs = seg_vs[j % 2]
        for g in range(_G_PER_CHUNK):
            o = j * _CHUNK + g * _L
            rows = lax.iota(jnp.int32, _L) + g * _L
            lane = flat_v[pl.ds(o, _L)] & 15
            z = mvoc_v[pl.ds(o, _L)]
            z = jnp.minimum(jnp.maximum(z, 0.0), 1.0)
            sf = win_v[pl.ds(o, _L)].astype(jnp.float32)
            d0 = z * _INV_SIGMA - sf * 1.25
            acc = plsc.load_gather(bufs[0], [rows, lane])
            for t in range(1, _NSTREAM):
                wk = plsc.load_gather(bufs[t], [rows, lane])
                diff = d0 - (0.625 + (t - 1) * 1.25)
                acc = acc + jnp.exp(diff * diff * -0.5) * wk
            out_v[pl.ds(o, _L)] = acc
        if j + 2 < _N_CHUNKS:
            copies.append(fire(j + 2))

    pltpu.sync_copy(out_v, out_hbm.at[pl.ds(base, _B_PER_W)])


def kernel(mvoc, day_idx, bucket_idx, weights):
    w_seg = weights.transpose(2, 0, 1).reshape(_SEG_ROWS, _L)
    out = _gauss_delta_sc(
        mvoc.reshape(_N),
        day_idx.reshape(_N),
        bucket_idx.reshape(_N),
        w_seg,
    )
    return out.reshape(_N, 1)


# speedup vs baseline: 6.4593x; 1.0076x over previous
import functools

import jax
import jax.numpy as jnp
from jax import lax
from jax.experimental import pallas as pl
from jax.experimental.pallas import tpu as pltpu
from jax.experimental.pallas import tpu_sc as plsc

_N_DAYS = 365
_N_BUCKETS = 1024
_N_RBF = 16
_N_BASES = 1 + _N_RBF
_N = 16384
_PLANE = _N_DAYS * _N_BUCKETS
_SEG_ROWS = _N_BASES * _PLANE // 16

_L = 16
_NC = 2
_NS = 16
_NW = _NC * _NS
_B_PER_W = _N // _NW
_CHUNK = 128
_N_CHUNKS = _B_PER_W // _CHUNK
_G_PER_CHUNK = _CHUNK // _L
_WIN = 6
_NSTREAM = 1 + _WIN

_INV_SIGMA = 20.0

_mesh = plsc.VectorSubcoreMesh(core_axis_name="c", subcore_axis_name="s")


@functools.partial(
    pl.kernel,
    mesh=_mesh,
    compiler_params=pltpu.CompilerParams(
        needs_layout_passes=False, use_tc_tiling_on_sc=False
    ),
    out_type=jax.ShapeDtypeStruct((_N,), jnp.float32),
    scratch_types=[
        pltpu.VMEM((_B_PER_W,), jnp.float32),
        pltpu.VMEM((_B_PER_W,), jnp.int32),
        pltpu.VMEM((_B_PER_W,), jnp.int32),
        pltpu.VMEM((_B_PER_W,), jnp.int32),
        pltpu.VMEM((_B_PER_W,), jnp.int32),
        pltpu.VMEM((_N_CHUNKS * _NSTREAM, _CHUNK), jnp.int32),
        [[pltpu.VMEM((_CHUNK, _L), jnp.float32)
          for _ in range(_NSTREAM)] for _ in range(_N_CHUNKS)],
        pltpu.VMEM((_B_PER_W,), jnp.float32),
        [pltpu.SemaphoreType.DMA for _ in range(_N_CHUNKS)],
    ],
)
def _gauss_delta_sc(mvoc_hbm, day_hbm, bucket_hbm, w_hbm, out_hbm,
                    mvoc_v, day_v, bucket_v, flat_v, win_v, sidx_v,
                    seg_vs, out_v, sems):
    wid = lax.axis_index("s") * _NC + lax.axis_index("c")
    base = wid * _B_PER_W

    pltpu.sync_copy(mvoc_hbm.at[pl.ds(base, _B_PER_W)], mvoc_v)
    pltpu.sync_copy(day_hbm.at[pl.ds(base, _B_PER_W)], day_v)
    pltpu.sync_copy(bucket_hbm.at[pl.ds(base, _B_PER_W)], bucket_v)

    copies = []
    for j in range(_N_CHUNKS):
        for g in range(_G_PER_CHUNK):
            o = j * _CHUNK + g * _L
            d = day_v[pl.ds(o, _L)]
            b = bucket_v[pl.ds(o, _L)]
            flat = d * _N_BUCKETS + b
            flat_v[pl.ds(o, _L)] = flat
            z = mvoc_v[pl.ds(o, _L)]
            z = jnp.minimum(jnp.maximum(z, 0.0), 1.0)
            m = (z * 16.0 - 0.5).astype(jnp.int32)
            s = jnp.minimum(jnp.maximum(m - 2, 0), 16 - _WIN)
            win_v[pl.ds(o, _L)] = s
            seg0 = flat >> 4
            for t in range(_NSTREAM):
                plane = (s + t) if t > 0 else jnp.zeros((_L,), jnp.int32)
                sidx_v[j * _NSTREAM + t, pl.ds(g * _L, _L)] = (
                    plane * (_PLANE // 16) + seg0)
        copies.append([pltpu.async_copy(
            w_hbm.at[sidx_v.at[j * _NSTREAM + t]], seg_vs[j][t], sems[j])
            for t in range(_NSTREAM)])

    for j in range(_N_CHUNKS):
        for c in copies[j]:
            c.wait()
        bufs = seg_vs[j]
        for g in range(_G_PER_CHUNK):
            o = j * _CHUNK + g * _L
            rows = lax.iota(jnp.int32, _L) + g * _L
            lane = flat_v[pl.ds(o, _L)] & 15
            z = mvoc_v[pl.ds(o, _L)]
            z = jnp.minimum(jnp.maximum(z, 0.0), 1.0)
            sf = win_v[pl.ds(o, _L)].astype(jnp.float32)
            d0 = z * _INV_SIGMA - sf * 1.25
            acc = plsc.load_gather(bufs[0], [rows, lane])
            for t in range(1, _NSTREAM):
                wk = plsc.load_gather(bufs[t], [rows, lane])
                diff = d0 - (0.625 + (t - 1) * 1.25)
                acc = acc + jnp.exp(diff * diff * -0.5) * wk
            out_v[pl.ds(o, _L)] = acc

    pltpu.sync_copy(out_v, out_hbm.at[pl.ds(base, _B_PER_W)])


def kernel(mvoc, day_idx, bucket_idx, weights):
    w_seg = weights.transpose(2, 0, 1).reshape(_SEG_ROWS, _L)
    out = _gauss_delta_sc(
        mvoc.reshape(_N),
        day_idx.reshape(_N),
        bucket_idx.reshape(_N),
        w_seg,
    )
    return out.reshape(_N, 1)
